# baseline (speedup 1.0000x reference)
import functools

import jax
import jax.numpy as jnp
from jax import lax
from jax.experimental import pallas as pl
from jax.experimental.pallas import tpu as pltpu
from jax.experimental.pallas import tpu_sc as plsc

_DEPTH = 50
_NP = 10240
_BM = 512
_NW = 32
_NODES_W = _NP // _NW
_NCH = 32
_E = 160000
_EW = _E // _NW
_HSLICE = _NP // 16


def _sc_embed_deg_body(tag_tab, sub_tab, tags_flat, subs_flat, offs_t, offs_s,
                       src_hbm, dst_hbm,
                       xp3_out, dego_out, degi_out,
                       tags_vm, subs_vm, idxt_vm, idxs_vm, rows_vm,
                       src_vm, dst_vm, ones_vm, zero_vm, idx_a, idx_b,
                       idx_ta, idx_tb, offt_vm, offs_vm, acc_do, acc_di, sem):
    c = lax.axis_index("c")
    s = lax.axis_index("s")
    w = c * 16 + s

    def fill_body(i, _):
        ones_vm[i] = jnp.ones((16,), jnp.float32)
        return 0
    lax.fori_loop(0, 128, fill_body, 0)

    def zfill_body(i, _):
        zero_vm[i] = jnp.zeros((16,), jnp.float32)
        return 0
    lax.fori_loop(0, _HSLICE, zfill_body, 0)

    zslice = pl.ds(s * _HSLICE, _HSLICE)
    pltpu.sync_copy(zero_vm, acc_do.at[zslice])
    pltpu.sync_copy(zero_vm, acc_di.at[zslice])
    @pl.when(s == 0)
    def _():
        pltpu.sync_copy(zero_vm.at[pl.ds(0, 16)], acc_do.at[pl.ds(_NP, 16)])
        pltpu.sync_copy(zero_vm.at[pl.ds(0, 16)], acc_di.at[pl.ds(_NP, 16)])
    plsc.subcore_barrier()

    ebase = w * _EW
    pltpu.sync_copy(src_hbm.at[pl.ds(ebase, _EW)], src_vm.at[pl.ds(0, _EW)])
    pltpu.sync_copy(dst_hbm.at[pl.ds(ebase, _EW)], dst_vm.at[pl.ds(0, _EW)])

    nfull = _EW // 128
    def deg_body(j, _):
        def cp_body(q, _):
            sl = pl.ds(q * 16, 16)
            idx_a[sl] = src_vm[pl.ds(j * 128 + q * 16, 16)]
            idx_b[sl] = dst_vm[pl.ds(j * 128 + q * 16, 16)]
            return 0
        lax.fori_loop(0, 8, cp_body, 0)
        pltpu.sync_copy(ones_vm, acc_do.at[idx_a], add=True)
        pltpu.sync_copy(ones_vm, acc_di.at[idx_b], add=True)
        return 0
    lax.fori_loop(0, nfull, deg_body, 0)

    lane = lax.iota(jnp.int32, 16)
    valid = lane < (_EW - nfull * 128)
    sv = src_vm[pl.ds(nfull * 128, 16)]
    dv = dst_vm[pl.ds(nfull * 128, 16)]
    idx_ta[pl.ds(0, 16)] = jnp.where(valid, sv, _NP)
    idx_tb[pl.ds(0, 16)] = jnp.where(valid, dv, _NP)
    pltpu.sync_copy(ones_vm.at[pl.ds(0, 16)], acc_do.at[idx_ta], add=True)
    pltpu.sync_copy(ones_vm.at[pl.ds(0, 16)], acc_di.at[idx_tb], add=True)
    plsc.subcore_barrier()

    pltpu.sync_copy(acc_do.at[zslice], dego_out.at[c, zslice])
    pltpu.sync_copy(acc_di.at[zslice], degi_out.at[c, zslice])

    pltpu.sync_copy(offs_t, offt_vm)
    pltpu.sync_copy(offs_s, offs_vm)
    nidx = _NCH * _DEPTH

    def chunk_body(j, _):
        nbase = w * _NODES_W + j * _NCH
        pltpu.sync_copy(tags_flat.at[pl.ds(nbase * _DEPTH, nidx)], tags_vm)
        pltpu.sync_copy(subs_flat.at[pl.ds(nbase * _DEPTH, nidx)], subs_vm)

        def idx_body(k, _):
            sl = pl.ds(k * 16, 16)
            idxt_vm[sl] = tags_vm[sl] + offt_vm[sl]
            idxs_vm[sl] = subs_vm[sl] + offs_vm[sl]
            return 0
        lax.fori_loop(0, nidx // 16, idx_body, 0)

        pltpu.async_copy(tag_tab.at[idxt_vm], rows_vm, sem).wait()
        pltpu.async_copy(sub_tab.at[idxs_vm], rows_vm, sem, add=True).wait()
        pltpu.sync_copy(rows_vm, xp3_out.at[pl.ds(nbase * _DEPTH, nidx)])
        return 0
    lax.fori_loop(0, _NODES_W // _NCH, chunk_body, 0)


def _sc_embed_deg(tag_tab, sub_tab, tags_flat, subs_flat, offs_t, offs_s,
                  src, dst):
    nidx = _NCH * _DEPTH
    mesh = plsc.VectorSubcoreMesh(core_axis_name="c", subcore_axis_name="s")
    f = pl.kernel(
        _sc_embed_deg_body,
        out_type=(
            jax.ShapeDtypeStruct((_NP * _DEPTH, 32), jnp.float32),
            jax.ShapeDtypeStruct((2, _NP, 16), jnp.float32),
            jax.ShapeDtypeStruct((2, _NP, 16), jnp.float32),
        ),
        mesh=mesh,
        compiler_params=pltpu.CompilerParams(use_tc_tiling_on_sc=False),
        scratch_types=[
            pltpu.VMEM((nidx,), jnp.int32),
            pltpu.VMEM((nidx,), jnp.int32),
            pltpu.VMEM((nidx,), jnp.int32),
            pltpu.VMEM((nidx,), jnp.int32),
            pltpu.VMEM((nidx, 32), jnp.float32),
            pltpu.VMEM((_EW + 120,), jnp.int32),
            pltpu.VMEM((_EW + 120,), jnp.int32),
            pltpu.VMEM((128, 16), jnp.float32),
            pltpu.VMEM((_HSLICE, 16), jnp.float32),
            pltpu.VMEM((128,), jnp.int32),
            pltpu.VMEM((128,), jnp.int32),
            pltpu.VMEM((16,), jnp.int32),
            pltpu.VMEM((16,), jnp.int32),
            pltpu.VMEM((nidx,), jnp.int32),
            pltpu.VMEM((nidx,), jnp.int32),
            pltpu.VMEM_SHARED((_NP + 16, 16), jnp.float32),
            pltpu.VMEM_SHARED((_NP + 16, 16), jnp.float32),
            pltpu.SemaphoreType.DMA,
        ],
    )
    return f(tag_tab, sub_tab, tags_flat, subs_flat, offs_t, offs_s, src, dst)


def _norm(deg):
    return jnp.where(deg > 0, jax.lax.rsqrt(jnp.maximum(deg, 1.0)), 0.0)


_Q = 64
_ET = _E // 16
_K = 128
_NFULL = _ET // _K


def _sc_segsum_body(hs_hbm, src_hbm, dst_hbm, agg_out,
                    src_vm, dst_vm, sidx_a, dloc_a, sidx_b, dloc_b,
                    sidx_t, dloc_t, rows_a, rows_b, rows_t, zrow_vm,
                    acc_sh, sem_a, sem_b):
    c = lax.axis_index("c")
    s = lax.axis_index("s")

    def zfill(i, _):
        def zin(k, _):
            zrow_vm[i, pl.ds(k * 16, 16)] = jnp.zeros((16,), jnp.float32)
            return 0
        lax.fori_loop(0, _Q // 16, zin, 0)
        return 0
    lax.fori_loop(0, 128, zfill, 0)

    ebase = s * _ET
    pltpu.sync_copy(src_hbm.at[pl.ds(ebase, _ET)], src_vm)
    pltpu.sync_copy(dst_hbm.at[pl.ds(ebase, _ET)], dst_vm)

    for p in range(2):
        q = 2 * p + c
        for z in range(5):
            pltpu.sync_copy(zrow_vm, acc_sh.at[pl.ds(s * 640 + z * 128, 128)])
        plsc.subcore_barrier()

        def build(k, sidx, dloc, nu):
            def bd(u, _):
                sl = pl.ds(u * 16, 16)
                esl = pl.ds(k * _K + u * 16, 16)
                sidx[sl] = src_vm[esl] * 4 + q
                dloc[sl] = dst_vm[esl]
                return 0
            lax.fori_loop(0, nu, bd, 0)

        build(0, sidx_a, dloc_a, _K // 16)
        pltpu.async_copy(hs_hbm.at[sidx_a], rows_a, sem_a)

        def pair(j, _):
            b = 2 * j + 1
            build(b, sidx_b, dloc_b, _K // 16)
            pltpu.async_copy(hs_hbm.at[sidx_b], rows_b, sem_b)
            pltpu.make_async_copy(hs_hbm.at[sidx_a], rows_a, sem_a).wait()
            pltpu.sync_copy(rows_a, acc_sh.at[dloc_a], add=True)

            @pl.when(j < _NFULL // 2 - 1)
            def _():
                build(b + 1, sidx_a, dloc_a, _K // 16)
                pltpu.async_copy(hs_hbm.at[sidx_a], rows_a, sem_a)
            pltpu.make_async_copy(hs_hbm.at[sidx_b], rows_b, sem_b).wait()
            pltpu.sync_copy(rows_b, acc_sh.at[dloc_b], add=True)
            return 0
        lax.fori_loop(0, _NFULL // 2, pair, 0)

        build(_NFULL, sidx_t, dloc_t, 1)
        pltpu.async_copy(hs_hbm.at[sidx_t], rows_t, sem_a).wait()
        pltpu.sync_copy(rows_t, acc_sh.at[dloc_t], add=True)
        plsc.subcore_barrier()

        pltpu.sync_copy(acc_sh.at[pl.ds(s * 640, 640)],
                        agg_out.at[q, pl.ds(s * 640, 640)])
        plsc.subcore_barrier()


def _sc_segsum(hs_flat, src, dst):
    mesh = plsc.VectorSubcoreMesh(core_axis_name="c", subcore_axis_name="s")
    f = pl.kernel(
        _sc_segsum_body,
        out_type=jax.ShapeDtypeStruct((4, _NP, _Q), jnp.float32),
        mesh=mesh,
        compiler_params=pltpu.CompilerParams(use_tc_tiling_on_sc=False),
        scratch_types=[
            pltpu.VMEM((_ET,), jnp.int32),
            pltpu.VMEM((_ET,), jnp.int32),
            pltpu.VMEM((_K,), jnp.int32),
            pltpu.VMEM((_K,), jnp.int32),
            pltpu.VMEM((_K,), jnp.int32),
            pltpu.VMEM((_K,), jnp.int32),
            pltpu.VMEM((16,), jnp.int32),
            pltpu.VMEM((16,), jnp.int32),
            pltpu.VMEM((_K, _Q), jnp.float32),
            pltpu.VMEM((_K, _Q), jnp.float32),
            pltpu.VMEM((16, _Q), jnp.float32),
            pltpu.VMEM((128, _Q), jnp.float32),
            pltpu.VMEM_SHARED((_NP, _Q), jnp.float32),
            pltpu.SemaphoreType.DMA,
            pltpu.SemaphoreType.DMA,
        ],
    )
    return f(hs_flat, src, dst)


def _sc_gather_body(agg_hbm, deg2_hbm, ids_hbm, offs_hbm,
                    ag_out, din_out,
                    ids_vm, off_vm, idx_vm, qidx_vm, rows_vm, dr_vm, sem):
    c = lax.axis_index("c")
    s = lax.axis_index("s")
    w = c * 16 + s
    rb = w * 32
    rsl = pl.ds(rb, 32)

    pltpu.sync_copy(ids_hbm.at[rsl], ids_vm)
    pltpu.sync_copy(offs_hbm.at[rsl], off_vm)
    for u in range(2):
        sl = pl.ds(u * 16, 16)
        idx_vm[sl] = ids_vm[sl] + off_vm[sl]
    for q in range(4):
        for u in range(2):
            sl = pl.ds(u * 16, 16)
            qidx_vm[sl] = idx_vm[sl] + q * _NP
        pltpu.async_copy(agg_hbm.at[qidx_vm], rows_vm, sem).wait()
        pltpu.sync_copy(rows_vm, ag_out.at[q, rsl])
    pltpu.async_copy(deg2_hbm.at[idx_vm], dr_vm, sem).wait()
    pltpu.sync_copy(dr_vm, din_out.at[0, rsl])
    for u in range(2):
        sl = pl.ds(u * 16, 16)
        qidx_vm[sl] = idx_vm[sl] + _NP
    pltpu.async_copy(deg2_hbm.at[qidx_vm], dr_vm, sem).wait()
    pltpu.sync_copy(dr_vm, din_out.at[1, rsl])


def _sc_gather(agg, deg2, ids_flat, offs):
    mesh = plsc.VectorSubcoreMesh(core_axis_name="c", subcore_axis_name="s")
    f = pl.kernel(
        _sc_gather_body,
        out_type=(
            jax.ShapeDtypeStruct((4, 1024, _Q), jnp.float32),
            jax.ShapeDtypeStruct((2, 1024, 16), jnp.float32),
        ),
        mesh=mesh,
        compiler_params=pltpu.CompilerParams(use_tc_tiling_on_sc=False),
        scratch_types=[
            pltpu.VMEM((32,), jnp.int32),
            pltpu.VMEM((32,), jnp.int32),
            pltpu.VMEM((32,), jnp.int32),
            pltpu.VMEM((32,), jnp.int32),
            pltpu.VMEM((32, _Q), jnp.float32),
            pltpu.VMEM((32, 16), jnp.float32),
            pltpu.SemaphoreType.DMA,
        ],
    )
    return f(agg, deg2, ids_flat, offs)


def _xp_mlp_body(x_ref, w1_ref, b1_ref, w2_ref, b2_ref, deg_ref, o_ref):
    x = x_ref[...].astype(jnp.bfloat16)
    t = jnp.dot(x, w1_ref[...].astype(jnp.bfloat16),
                preferred_element_type=jnp.float32)
    t = jnp.maximum(t + b1_ref[...][None, :], 0.0)
    y = jnp.dot(t.astype(jnp.bfloat16), w2_ref[...].astype(jnp.bfloat16),
                preferred_element_type=jnp.float32)
    y = y + b2_ref[...][None, :]
    o_ref[...] = y * _norm(deg_ref[0, :] + deg_ref[1, :])[:, None]


def _xp_mlp(xp_raw, w1, b1, w2, b2, deg_out):
    n, k = xp_raw.shape
    grid = (n // _BM,)
    return pl.pallas_call(
        _xp_mlp_body,
        grid=grid,
        in_specs=[
            pl.BlockSpec((_BM, k), lambda i: (i, 0)),
            pl.BlockSpec(w1.shape, lambda i: (0, 0)),
            pl.BlockSpec(b1.shape, lambda i: (0,)),
            pl.BlockSpec(w2.shape, lambda i: (0, 0)),
            pl.BlockSpec(b2.shape, lambda i: (0,)),
            pl.BlockSpec((2, _BM), lambda i: (0, i)),
        ],
        out_specs=pl.BlockSpec((_BM, w2.shape[1]), lambda i: (i, 0)),
        out_shape=jax.ShapeDtypeStruct((n, w2.shape[1]), jnp.float32),
    )(xp_raw, w1, b1, w2, b2, deg_out)


def _gcn_body(agg_ref, din_ref, dout_ref, w_ref, b_ref, o_ref):
    agg = jnp.concatenate([agg_ref[i] for i in range(4)], axis=-1)
    a = agg * _norm(din_ref[0, :] + din_ref[1, :])[:, None]
    h = jnp.dot(a, w_ref[...], preferred_element_type=jnp.float32)
    h = jnp.maximum(h + b_ref[...][None, :], 0.0)
    o_ref[...] = h * _norm(dout_ref[0, :] + dout_ref[1, :])[:, None]


def _gcn_layer(agg4, deg_in, deg_out, w, b):
    _, n, dq = agg4.shape
    bm = 1024
    return pl.pallas_call(
        _gcn_body,
        grid=(n // bm,),
        in_specs=[
            pl.BlockSpec((4, bm, dq), lambda i: (0, i, 0)),
            pl.BlockSpec((2, bm), lambda i: (0, i)),
            pl.BlockSpec((2, bm), lambda i: (0, i)),
            pl.BlockSpec(w.shape, lambda i: (0, 0)),
            pl.BlockSpec(b.shape, lambda i: (0,)),
        ],
        out_specs=pl.BlockSpec((bm, 4 * dq), lambda i: (i, 0)),
        out_shape=jax.ShapeDtypeStruct((n, 4 * dq), jnp.float32),
    )(agg4, deg_in, deg_out, w, b)


def _final_body(te_ref, ag_ref, din_ref, wg_ref, bg_ref,
                w1_ref, b1_ref, w2_ref, b2_ref, w3_ref, b3_ref, o_ref):
    ag = jnp.concatenate([ag_ref[i] for i in range(4)], axis=-1)
    tnx = ag * _norm(din_ref[0, :] + din_ref[1, :])[:, None]
    tnx = jnp.dot(tnx, wg_ref[...], preferred_element_type=jnp.float32)
    tnx = tnx + bg_ref[...][None, :]

    def l2(x):
        n = jnp.sqrt(jnp.sum(x * x, axis=1, keepdims=True))
        return x / jnp.maximum(n, 1e-12)

    feat = jnp.concatenate([l2(te_ref[...]), l2(tnx)], axis=1)
    z = jnp.dot(feat, w1_ref[...], preferred_element_type=jnp.float32)
    z = jnp.maximum(z + b1_ref[...][None, :], 0.0)
    z = jnp.dot(z, w2_ref[...], preferred_element_type=jnp.float32)
    z = jnp.maximum(z + b2_ref[...][None, :], 0.0)
    z = jnp.dot(z, w3_ref[...], preferred_element_type=jnp.float32)
    o_ref[...] = z + b3_ref[...][None, :]


def _final_mlp(te, ag, din, wg, bg, w1, b1, w2, b2, w3, b3):
    m = te.shape[0]
    args = (te, ag, din, wg, bg, w1, b1, w2, b2, w3, b3)
    return pl.pallas_call(
        _final_body,
        grid=(1,),
        in_specs=[pl.BlockSpec(a.shape, functools.partial(lambda nd, i: (0,) * nd, a.ndim))
                  for a in args],
        out_specs=pl.BlockSpec((m, 8), lambda i: (0, 0)),
        out_shape=jax.ShapeDtypeStruct((m, 8), jnp.float32),
    )(*args)


def kernel(ids, text_embeddings, xpath_tags_seq, xpath_subs_seq, edge_index,
           tag_emb, subs_emb, W_inner, b_inner, W_emb, b_emb,
           W_g1, b_g1, W_g2, b_g2, W_g3, b_g3,
           W_m1, b_m1, W_m2, b_m2, W_m3, b_m3):
    n = xpath_tags_seq.shape[0]
    tags = jnp.squeeze(xpath_tags_seq, axis=1).astype(jnp.int32)
    subs = jnp.squeeze(xpath_subs_seq, axis=1).astype(jnp.int32)
    src = edge_index[0].astype(jnp.int32)
    dst = edge_index[1].astype(jnp.int32)

    tags_flat = jnp.pad(tags, ((0, _NP - n), (0, 0))).reshape(-1)
    subs_flat = jnp.pad(subs, ((0, _NP - n), (0, 0))).reshape(-1)
    dseq = jnp.arange(_NCH * _DEPTH, dtype=jnp.int32) % _DEPTH
    offs_t = dseq * tag_emb.shape[1]
    offs_s = dseq * subs_emb.shape[1]
    tag_tab = tag_emb.reshape(-1, tag_emb.shape[2])
    sub_tab = subs_emb.reshape(-1, subs_emb.shape[2])
    xp3, dego3, degi3 = _sc_embed_deg(tag_tab, sub_tab, tags_flat, subs_flat,
                                      offs_t, offs_s, src, dst)
    xp_raw = xp3.reshape(_NP, _DEPTH * 32)
    dego = dego3[:, :, 0]
    degi = degi3[:, :, 0]

    hs = _xp_mlp(xp_raw, W_inner, b_inner, W_emb, b_emb, dego)

    agg1 = _sc_segsum(hs.reshape(_NP * 4, _Q), src, dst)
    hs1 = _gcn_layer(agg1, degi, dego, W_g1, b_g1)
    agg2 = _sc_segsum(hs1.reshape(_NP * 4, _Q), src, dst)
    hs2 = _gcn_layer(agg2, degi, dego, W_g2, b_g2)
    agg3 = _sc_segsum(hs2.reshape(_NP * 4, _Q), src, dst)

    n_graphs, per = ids.shape
    m = n_graphs * per
    mp = 1024
    ids_flat = jnp.pad(ids.reshape(-1).astype(jnp.int32), (0, mp - m))
    ar = jnp.arange(mp, dtype=jnp.int32)
    offs_g = jnp.where(ar < m, (ar // per) * (n // n_graphs), 0)
    ag, din3 = _sc_gather(agg3.reshape(4 * _NP, _Q),
                          degi3.reshape(2 * _NP, 16), ids_flat, offs_g)
    din_g = din3[:, :, 0]
    te = jnp.pad(text_embeddings, ((0, mp - m), (0, 0)))

    out = _final_mlp(te, ag, din_g, W_g3, b_g3, W_m1, b_m1, W_m2, b_m2, W_m3, b_m3)
    return out[:m]


# speedup vs baseline: 1.0014x; 1.0014x over previous
import functools

import jax
import jax.numpy as jnp
from jax import lax
from jax.experimental import pallas as pl
from jax.experimental.pallas import tpu as pltpu
from jax.experimental.pallas import tpu_sc as plsc

_DEPTH = 50
_NP = 10240
_BM = 512
_NW = 32
_NODES_W = _NP // _NW
_NCH = 32
_E = 160000
_EW = _E // _NW
_HSLICE = _NP // 16


def _sc_embed_deg_body(tag_tab, sub_tab, tags_flat, subs_flat, offs_t, offs_s,
                       src_hbm, dst_hbm,
                       xp3_out, dego_out, degi_out,
                       tags_vm, subs_vm, idxt_vm, idxs_vm, rows_vm,
                       src_vm, dst_vm, ones_vm, zero_vm, idx_a, idx_b,
                       idx_ta, idx_tb, offt_vm, offs_vm, acc_do, acc_di, sem):
    c = lax.axis_index("c")
    s = lax.axis_index("s")
    w = c * 16 + s

    def fill_body(i, _):
        ones_vm[i] = jnp.ones((16,), jnp.float32)
        return 0
    lax.fori_loop(0, 128, fill_body, 0)

    def zfill_body(i, _):
        zero_vm[i] = jnp.zeros((16,), jnp.float32)
        return 0
    lax.fori_loop(0, _HSLICE, zfill_body, 0)

    zslice = pl.ds(s * _HSLICE, _HSLICE)
    pltpu.sync_copy(zero_vm, acc_do.at[zslice])
    pltpu.sync_copy(zero_vm, acc_di.at[zslice])
    @pl.when(s == 0)
    def _():
        pltpu.sync_copy(zero_vm.at[pl.ds(0, 16)], acc_do.at[pl.ds(_NP, 16)])
        pltpu.sync_copy(zero_vm.at[pl.ds(0, 16)], acc_di.at[pl.ds(_NP, 16)])
    plsc.subcore_barrier()

    ebase = w * _EW
    pltpu.sync_copy(src_hbm.at[pl.ds(ebase, _EW)], src_vm.at[pl.ds(0, _EW)])
    pltpu.sync_copy(dst_hbm.at[pl.ds(ebase, _EW)], dst_vm.at[pl.ds(0, _EW)])

    nfull = _EW // 128
    def deg_body(j, _):
        def cp_body(q, _):
            sl = pl.ds(q * 16, 16)
            idx_a[sl] = src_vm[pl.ds(j * 128 + q * 16, 16)]
            idx_b[sl] = dst_vm[pl.ds(j * 128 + q * 16, 16)]
            return 0
        lax.fori_loop(0, 8, cp_body, 0)
        pltpu.sync_copy(ones_vm, acc_do.at[idx_a], add=True)
        pltpu.sync_copy(ones_vm, acc_di.at[idx_b], add=True)
        return 0
    lax.fori_loop(0, nfull, deg_body, 0)

    lane = lax.iota(jnp.int32, 16)
    valid = lane < (_EW - nfull * 128)
    sv = src_vm[pl.ds(nfull * 128, 16)]
    dv = dst_vm[pl.ds(nfull * 128, 16)]
    idx_ta[pl.ds(0, 16)] = jnp.where(valid, sv, _NP)
    idx_tb[pl.ds(0, 16)] = jnp.where(valid, dv, _NP)
    pltpu.sync_copy(ones_vm.at[pl.ds(0, 16)], acc_do.at[idx_ta], add=True)
    pltpu.sync_copy(ones_vm.at[pl.ds(0, 16)], acc_di.at[idx_tb], add=True)
    plsc.subcore_barrier()

    pltpu.sync_copy(acc_do.at[zslice], dego_out.at[c, zslice])
    pltpu.sync_copy(acc_di.at[zslice], degi_out.at[c, zslice])

    pltpu.sync_copy(offs_t, offt_vm)
    pltpu.sync_copy(offs_s, offs_vm)
    nidx = _NCH * _DEPTH

    def chunk_body(j, _):
        nbase = w * _NODES_W + j * _NCH
        pltpu.sync_copy(tags_flat.at[pl.ds(nbase * _DEPTH, nidx)], tags_vm)
        pltpu.sync_copy(subs_flat.at[pl.ds(nbase * _DEPTH, nidx)], subs_vm)

        def idx_body(k, _):
            sl = pl.ds(k * 16, 16)
            idxt_vm[sl] = tags_vm[sl] + offt_vm[sl]
            idxs_vm[sl] = subs_vm[sl] + offs_vm[sl]
            return 0
        lax.fori_loop(0, nidx // 16, idx_body, 0)

        pltpu.async_copy(tag_tab.at[idxt_vm], rows_vm, sem).wait()
        pltpu.async_copy(sub_tab.at[idxs_vm], rows_vm, sem, add=True).wait()
        pltpu.sync_copy(rows_vm, xp3_out.at[pl.ds(nbase * _DEPTH, nidx)])
        return 0
    lax.fori_loop(0, _NODES_W // _NCH, chunk_body, 0)


def _sc_embed_deg(tag_tab, sub_tab, tags_flat, subs_flat, offs_t, offs_s,
                  src, dst):
    nidx = _NCH * _DEPTH
    mesh = plsc.VectorSubcoreMesh(core_axis_name="c", subcore_axis_name="s")
    f = pl.kernel(
        _sc_embed_deg_body,
        out_type=(
            jax.ShapeDtypeStruct((_NP * _DEPTH, 32), jnp.float32),
            jax.ShapeDtypeStruct((2, _NP, 16), jnp.float32),
            jax.ShapeDtypeStruct((2, _NP, 16), jnp.float32),
        ),
        mesh=mesh,
        compiler_params=pltpu.CompilerParams(use_tc_tiling_on_sc=False),
        scratch_types=[
            pltpu.VMEM((nidx,), jnp.int32),
            pltpu.VMEM((nidx,), jnp.int32),
            pltpu.VMEM((nidx,), jnp.int32),
            pltpu.VMEM((nidx,), jnp.int32),
            pltpu.VMEM((nidx, 32), jnp.float32),
            pltpu.VMEM((_EW + 120,), jnp.int32),
            pltpu.VMEM((_EW + 120,), jnp.int32),
            pltpu.VMEM((128, 16), jnp.float32),
            pltpu.VMEM((_HSLICE, 16), jnp.float32),
            pltpu.VMEM((128,), jnp.int32),
            pltpu.VMEM((128,), jnp.int32),
            pltpu.VMEM((16,), jnp.int32),
            pltpu.VMEM((16,), jnp.int32),
            pltpu.VMEM((nidx,), jnp.int32),
            pltpu.VMEM((nidx,), jnp.int32),
            pltpu.VMEM_SHARED((_NP + 16, 16), jnp.float32),
            pltpu.VMEM_SHARED((_NP + 16, 16), jnp.float32),
            pltpu.SemaphoreType.DMA,
        ],
    )
    return f(tag_tab, sub_tab, tags_flat, subs_flat, offs_t, offs_s, src, dst)


def _norm(deg):
    return jnp.where(deg > 0, jax.lax.rsqrt(jnp.maximum(deg, 1.0)), 0.0)


_Q = 64
_ET = _E // 16
_K = 128
_NFULL = _ET // _K


def _sc_segsum_body(final, *refs):
    if final:
        (hs_hbm, src_hbm, dst_hbm, deg2_hbm, ids_hbm, offs_hbm,
         agg_out, ag_out, din_out,
         src_vm, dst_vm, sidx_a, dloc_a, sidx_b, dloc_b,
         sidx_t, dloc_t, rows_a, rows_b, rows_t, zrow_vm,
         gi_vm, go_vm, gq_vm, grows_vm, gdr_vm,
         acc_sh, sem_a, sem_b) = refs
    else:
        (hs_hbm, src_hbm, dst_hbm, agg_out,
         src_vm, dst_vm, sidx_a, dloc_a, sidx_b, dloc_b,
         sidx_t, dloc_t, rows_a, rows_b, rows_t, zrow_vm,
         acc_sh, sem_a, sem_b) = refs
    c = lax.axis_index("c")
    s = lax.axis_index("s")

    def zfill(i, _):
        def zin(k, _):
            zrow_vm[i, pl.ds(k * 16, 16)] = jnp.zeros((16,), jnp.float32)
            return 0
        lax.fori_loop(0, _Q // 16, zin, 0)
        return 0
    lax.fori_loop(0, 128, zfill, 0)

    ebase = s * _ET
    pltpu.sync_copy(src_hbm.at[pl.ds(ebase, _ET)], src_vm)
    pltpu.sync_copy(dst_hbm.at[pl.ds(ebase, _ET)], dst_vm)

    for p in range(2):
        q = 2 * p + c
        for z in range(5):
            pltpu.sync_copy(zrow_vm, acc_sh.at[pl.ds(s * 640 + z * 128, 128)])
        plsc.subcore_barrier()

        def build(k, sidx, dloc, nu):
            def bd(u, _):
                sl = pl.ds(u * 16, 16)
                esl = pl.ds(k * _K + u * 16, 16)
                sidx[sl] = src_vm[esl] * 4 + q
                dloc[sl] = dst_vm[esl]
                return 0
            lax.fori_loop(0, nu, bd, 0)

        build(0, sidx_a, dloc_a, _K // 16)
        pltpu.async_copy(hs_hbm.at[sidx_a], rows_a, sem_a)

        def pair(j, _):
            b = 2 * j + 1
            build(b, sidx_b, dloc_b, _K // 16)
            pltpu.async_copy(hs_hbm.at[sidx_b], rows_b, sem_b)
            pltpu.make_async_copy(hs_hbm.at[sidx_a], rows_a, sem_a).wait()
            pltpu.sync_copy(rows_a, acc_sh.at[dloc_a], add=True)

            @pl.when(j < _NFULL // 2 - 1)
            def _():
                build(b + 1, sidx_a, dloc_a, _K // 16)
                pltpu.async_copy(hs_hbm.at[sidx_a], rows_a, sem_a)
            pltpu.make_async_copy(hs_hbm.at[sidx_b], rows_b, sem_b).wait()
            pltpu.sync_copy(rows_b, acc_sh.at[dloc_b], add=True)
            return 0
        lax.fori_loop(0, _NFULL // 2, pair, 0)

        build(_NFULL, sidx_t, dloc_t, 1)
        pltpu.async_copy(hs_hbm.at[sidx_t], rows_t, sem_a).wait()
        pltpu.sync_copy(rows_t, acc_sh.at[dloc_t], add=True)
        plsc.subcore_barrier()

        pltpu.sync_copy(acc_sh.at[pl.ds(s * 640, 640)],
                        agg_out.at[pl.ds(q * _NP + s * 640, 640)])
        plsc.subcore_barrier()

        if final:
            rb = s * 64
            if p == 0:
                pltpu.sync_copy(ids_hbm.at[pl.ds(rb, 64)], gi_vm)
                pltpu.sync_copy(offs_hbm.at[pl.ds(rb, 64)], go_vm)
                for u in range(4):
                    sl = pl.ds(u * 16, 16)
                    gi_vm[sl] = gi_vm[sl] + go_vm[sl]
                for u in range(4):
                    sl = pl.ds(u * 16, 16)
                    gq_vm[sl] = gi_vm[sl] + c * _NP
                pltpu.async_copy(deg2_hbm.at[gq_vm], gdr_vm, sem_a).wait()
                pltpu.sync_copy(gdr_vm, din_out.at[c, pl.ds(rb, 64)])
            for u in range(4):
                sl = pl.ds(u * 16, 16)
                gq_vm[sl] = gi_vm[sl] + q * _NP
            pltpu.async_copy(agg_out.at[gq_vm], grows_vm, sem_a).wait()
            pltpu.sync_copy(grows_vm, ag_out.at[pl.ds(q * 1024 + rb, 64)])


def _make_sc_segsum(final):
    mesh = plsc.VectorSubcoreMesh(core_axis_name="c", subcore_axis_name="s")
    out_type = [jax.ShapeDtypeStruct((4 * _NP, _Q), jnp.float32)]
    scratch = [
        pltpu.VMEM((_ET,), jnp.int32),
        pltpu.VMEM((_ET,), jnp.int32),
        pltpu.VMEM((_K,), jnp.int32),
        pltpu.VMEM((_K,), jnp.int32),
        pltpu.VMEM((_K,), jnp.int32),
        pltpu.VMEM((_K,), jnp.int32),
        pltpu.VMEM((16,), jnp.int32),
        pltpu.VMEM((16,), jnp.int32),
        pltpu.VMEM((_K, _Q), jnp.float32),
        pltpu.VMEM((_K, _Q), jnp.float32),
        pltpu.VMEM((16, _Q), jnp.float32),
        pltpu.VMEM((128, _Q), jnp.float32),
    ]
    if final:
        out_type += [jax.ShapeDtypeStruct((4 * 1024, _Q), jnp.float32),
                     jax.ShapeDtypeStruct((2, 1024, 16), jnp.float32)]
        scratch += [
            pltpu.VMEM((64,), jnp.int32),
            pltpu.VMEM((64,), jnp.int32),
            pltpu.VMEM((64,), jnp.int32),
            pltpu.VMEM((64, _Q), jnp.float32),
            pltpu.VMEM((64, 16), jnp.float32),
        ]
    scratch += [
        pltpu.VMEM_SHARED((_NP, _Q), jnp.float32),
        pltpu.SemaphoreType.DMA,
        pltpu.SemaphoreType.DMA,
    ]
    return pl.kernel(
        functools.partial(_sc_segsum_body, final),
        out_type=tuple(out_type),
        mesh=mesh,
        compiler_params=pltpu.CompilerParams(use_tc_tiling_on_sc=False),
        scratch_types=scratch,
    )


def _sc_segsum_mid(*args):
    return _make_sc_segsum(False)(*args)


def _sc_segsum_fin(*args):
    return _make_sc_segsum(True)(*args)


def _xp_mlp_body(x_ref, w1_ref, b1_ref, w2_ref, b2_ref, deg_ref, o_ref):
    x = x_ref[...]
    t = jnp.dot(x, w1_ref[...], preferred_element_type=jnp.float32)
    t = jnp.maximum(t + b1_ref[...][None, :], 0.0)
    y = jnp.dot(t, w2_ref[...], preferred_element_type=jnp.float32)
    y = y + b2_ref[...][None, :]
    o_ref[...] = y * _norm(deg_ref[0, :] + deg_ref[1, :])[:, None]


def _xp_mlp(xp_raw, w1, b1, w2, b2, deg_out):
    n, k = xp_raw.shape
    grid = (n // _BM,)
    return pl.pallas_call(
        _xp_mlp_body,
        grid=grid,
        in_specs=[
            pl.BlockSpec((_BM, k), lambda i: (i, 0)),
            pl.BlockSpec(w1.shape, lambda i: (0, 0)),
            pl.BlockSpec(b1.shape, lambda i: (0,)),
            pl.BlockSpec(w2.shape, lambda i: (0, 0)),
            pl.BlockSpec(b2.shape, lambda i: (0,)),
            pl.BlockSpec((2, _BM), lambda i: (0, i)),
        ],
        out_specs=pl.BlockSpec((_BM, w2.shape[1]), lambda i: (i, 0)),
        out_shape=jax.ShapeDtypeStruct((n, w2.shape[1]), jnp.float32),
    )(xp_raw, w1, b1, w2, b2, deg_out)


def _gcn_body(agg_ref, din_ref, dout_ref, w_ref, b_ref, o_ref):
    agg = jnp.concatenate([agg_ref[i] for i in range(4)], axis=-1)
    a = agg * _norm(din_ref[0, :] + din_ref[1, :])[:, None]
    h = jnp.dot(a, w_ref[...], preferred_element_type=jnp.float32)
    h = jnp.maximum(h + b_ref[...][None, :], 0.0)
    o_ref[...] = h * _norm(dout_ref[0, :] + dout_ref[1, :])[:, None]


def _gcn_layer(agg4, deg_in, deg_out, w, b):
    _, n, dq = agg4.shape
    bm = 1024
    return pl.pallas_call(
        _gcn_body,
        grid=(n // bm,),
        in_specs=[
            pl.BlockSpec((4, bm, dq), lambda i: (0, i, 0)),
            pl.BlockSpec((2, bm), lambda i: (0, i)),
            pl.BlockSpec((2, bm), lambda i: (0, i)),
            pl.BlockSpec(w.shape, lambda i: (0, 0)),
            pl.BlockSpec(b.shape, lambda i: (0,)),
        ],
        out_specs=pl.BlockSpec((bm, 4 * dq), lambda i: (i, 0)),
        out_shape=jax.ShapeDtypeStruct((n, 4 * dq), jnp.float32),
    )(agg4, deg_in, deg_out, w, b)


def _final_body(te_ref, ag_ref, din_ref, wg_ref, bg_ref,
                w1_ref, b1_ref, w2_ref, b2_ref, w3_ref, b3_ref, o_ref):
    ag = jnp.concatenate([ag_ref[i] for i in range(4)], axis=-1)
    tnx = ag * _norm(din_ref[0, :] + din_ref[1, :])[:, None]
    tnx = jnp.dot(tnx, wg_ref[...], preferred_element_type=jnp.float32)
    tnx = tnx + bg_ref[...][None, :]

    def l2(x):
        n = jnp.sqrt(jnp.sum(x * x, axis=1, keepdims=True))
        return x / jnp.maximum(n, 1e-12)

    feat = jnp.concatenate([l2(te_ref[...]), l2(tnx)], axis=1)
    z = jnp.dot(feat, w1_ref[...], preferred_element_type=jnp.float32)
    z = jnp.maximum(z + b1_ref[...][None, :], 0.0)
    z = jnp.dot(z, w2_ref[...], preferred_element_type=jnp.float32)
    z = jnp.maximum(z + b2_ref[...][None, :], 0.0)
    z = jnp.dot(z, w3_ref[...], preferred_element_type=jnp.float32)
    o_ref[...] = z + b3_ref[...][None, :]


def _final_mlp(te, ag, din, wg, bg, w1, b1, w2, b2, w3, b3):
    m = te.shape[0]
    args = (te, ag, din, wg, bg, w1, b1, w2, b2, w3, b3)
    return pl.pallas_call(
        _final_body,
        grid=(1,),
        in_specs=[pl.BlockSpec(a.shape, functools.partial(lambda nd, i: (0,) * nd, a.ndim))
                  for a in args],
        out_specs=pl.BlockSpec((m, 8), lambda i: (0, 0)),
        out_shape=jax.ShapeDtypeStruct((m, 8), jnp.float32),
    )(*args)


def kernel(ids, text_embeddings, xpath_tags_seq, xpath_subs_seq, edge_index,
           tag_emb, subs_emb, W_inner, b_inner, W_emb, b_emb,
           W_g1, b_g1, W_g2, b_g2, W_g3, b_g3,
           W_m1, b_m1, W_m2, b_m2, W_m3, b_m3):
    n = xpath_tags_seq.shape[0]
    tags = jnp.squeeze(xpath_tags_seq, axis=1).astype(jnp.int32)
    subs = jnp.squeeze(xpath_subs_seq, axis=1).astype(jnp.int32)
    src = edge_index[0].astype(jnp.int32)
    dst = edge_index[1].astype(jnp.int32)

    tags_flat = jnp.pad(tags, ((0, _NP - n), (0, 0))).reshape(-1)
    subs_flat = jnp.pad(subs, ((0, _NP - n), (0, 0))).reshape(-1)
    dseq = jnp.arange(_NCH * _DEPTH, dtype=jnp.int32) % _DEPTH
    offs_t = dseq * tag_emb.shape[1]
    offs_s = dseq * subs_emb.shape[1]
    tag_tab = tag_emb.reshape(-1, tag_emb.shape[2])
    sub_tab = subs_emb.reshape(-1, subs_emb.shape[2])
    xp3, dego3, degi3 = _sc_embed_deg(tag_tab, sub_tab, tags_flat, subs_flat,
                                      offs_t, offs_s, src, dst)
    xp_raw = xp3.reshape(_NP, _DEPTH * 32)
    dego = dego3[:, :, 0]
    degi = degi3[:, :, 0]

    hs = _xp_mlp(xp_raw, W_inner, b_inner, W_emb, b_emb, dego)

    n_graphs, per = ids.shape
    m = n_graphs * per
    mp = 1024
    ids_flat = jnp.pad(ids.reshape(-1).astype(jnp.int32), (0, mp - m))
    ar = jnp.arange(mp, dtype=jnp.int32)
    offs_g = jnp.where(ar < m, (ar // per) * (n // n_graphs), 0)

    (agg1,) = _sc_segsum_mid(hs.reshape(_NP * 4, _Q), src, dst)
    hs1 = _gcn_layer(agg1.reshape(4, _NP, _Q), degi, dego, W_g1, b_g1)
    (agg2,) = _sc_segsum_mid(hs1.reshape(_NP * 4, _Q), src, dst)
    hs2 = _gcn_layer(agg2.reshape(4, _NP, _Q), degi, dego, W_g2, b_g2)
    _, agv, din3 = _sc_segsum_fin(hs2.reshape(_NP * 4, _Q), src, dst,
                                  degi3.reshape(2 * _NP, 16),
                                  ids_flat, offs_g)
    ag = agv.reshape(4, 1024, _Q)
    din_g = din3[:, :, 0]
    te = jnp.pad(text_embeddings, ((0, mp - m), (0, 0)))

    out = _final_mlp(te, ag, din_g, W_g3, b_g3, W_m1, b_m1, W_m2, b_m2, W_m3, b_m3)
    return out[:m]
